# Initial kernel scaffold; baseline (speedup 1.0000x reference)
import jax
import jax.numpy as jnp
from jax.experimental import pallas as pl


def kernel(asp_scores, opn_scores, span_reprs, null_asp_repr, null_opn_repr, W1, b1, W_cat, b_cat, W_sent, b_sent):
    raise NotImplementedError("write your pallas kernel here")


# speedup vs baseline: 2.5978x; 2.5978x over previous
import functools

import jax
import jax.numpy as jnp
from jax import lax
from jax.experimental import pallas as pl
from jax.experimental.pallas import tpu as pltpu
from jax.experimental.pallas import tpu_sc as plsc

K = 64
N = 4096
D = 512
B = 8
NPAD = 72
NROW = 2 * B


def _topk_body(s_ref, vals_ref, ids_ref):
    scores = s_ref[...]
    col = lax.broadcasted_iota(jnp.int32, (NROW, N), 1)
    lane_k = lax.broadcasted_iota(jnp.int32, (NROW, K), 1)

    def step(k, carry):
        sc, vals, ids = carry
        m = jnp.max(sc, axis=1, keepdims=True)
        cand = jnp.where(sc >= m, col, jnp.int32(N))
        idx = jnp.min(cand, axis=1, keepdims=True)
        vals = jnp.where(lane_k == k, m, vals)
        ids = jnp.where(lane_k == k, idx, ids)
        sc = jnp.where(col == idx, -jnp.inf, sc)
        return sc, vals, ids

    vals0 = jnp.zeros((NROW, K), jnp.float32)
    ids0 = jnp.zeros((NROW, K), jnp.int32)
    _, vals, ids = lax.fori_loop(0, K, step, (scores, vals0, ids0))
    vals_ref[...] = vals
    row = lax.broadcasted_iota(jnp.int32, (NROW, K), 0)
    ids_ref[...] = ids + (row % B) * N


def _topk(scores_stacked):
    return pl.pallas_call(
        _topk_body,
        out_shape=[
            jax.ShapeDtypeStruct((NROW, K), jnp.float32),
            jax.ShapeDtypeStruct((NROW, K), jnp.int32),
        ],
    )(scores_stacked)


def _sc_gather(table, idx_flat):
    n_idx = NROW * K
    nw = 32
    bpw = n_idx // nw
    mesh = plsc.VectorSubcoreMesh(core_axis_name="c", subcore_axis_name="s")

    @functools.partial(
        pl.kernel,
        mesh=mesh,
        out_type=jax.ShapeDtypeStruct((n_idx, D), jnp.float32),
        scratch_types=[
            pltpu.VMEM((bpw,), jnp.int32),
            pltpu.VMEM((bpw, D), jnp.float32),
            pltpu.SemaphoreType.DMA,
        ],
    )
    def gk(table_hbm, idx_hbm, out_hbm, idx_v, rows_v, sem):
        wid = lax.axis_index("s") * 2 + lax.axis_index("c")
        base = wid * bpw
        pltpu.sync_copy(idx_hbm.at[pl.ds(base, bpw)], idx_v)
        pltpu.async_copy(table_hbm.at[idx_v], rows_v, sem).wait()
        pltpu.sync_copy(rows_v, out_hbm.at[pl.ds(base, bpw)])

    return gk(table, idx_flat)


def _mlp_body(a_ref, o_ref, w1_ref, b1_ref, wc_ref, bc_ref, out_ref):
    a = a_ref[0]
    o = o_ref[0]
    pa = jnp.dot(a, w1_ref[:D, :], preferred_element_type=jnp.float32)
    po = jnp.dot(o, w1_ref[D:, :], preferred_element_type=jnp.float32)
    po = po + b1_ref[...]
    h = jax.nn.relu(pa[:, None, :] + po[None, :, :])
    h2 = h.reshape(NPAD * NPAD, D)
    out = jnp.dot(h2, wc_ref[...], preferred_element_type=jnp.float32)
    out_ref[0] = out + bc_ref[...]


def _mlp(asp72, opn72, w1, b1r, wc, bcr):
    return pl.pallas_call(
        _mlp_body,
        grid=(B,),
        in_specs=[
            pl.BlockSpec((1, NPAD, D), lambda b: (b, 0, 0)),
            pl.BlockSpec((1, NPAD, D), lambda b: (b, 0, 0)),
            pl.BlockSpec((2 * D, D), lambda b: (0, 0)),
            pl.BlockSpec((1, D), lambda b: (0, 0)),
            pl.BlockSpec((D, 16), lambda b: (0, 0)),
            pl.BlockSpec((1, 16), lambda b: (0, 0)),
        ],
        out_specs=pl.BlockSpec((1, NPAD * NPAD, 16), lambda b: (b, 0, 0)),
        out_shape=jax.ShapeDtypeStruct((B, NPAD * NPAD, 16), jnp.float32),
    )(asp72, opn72, w1, b1r, wc, bcr)


def kernel(asp_scores, opn_scores, span_reprs, null_asp_repr, null_opn_repr,
           W1, b1, W_cat, b_cat, W_sent, b_sent):
    scores = jnp.concatenate([asp_scores, opn_scores], axis=0)
    vals, ids = _topk(scores)
    asp_topk_scores = vals[:B]
    opn_topk_scores = vals[B:]

    table = span_reprs.reshape(B * N, D)
    rows = _sc_gather(table, ids.reshape(-1))
    rows = rows.reshape(2, B, K, D)

    pad = jnp.zeros((B, NPAD - K - 1, D), jnp.float32)
    asp72 = jnp.concatenate([rows[0], null_asp_repr[:, None, :], pad], axis=1)
    opn72 = jnp.concatenate([rows[1], null_opn_repr[:, None, :], pad], axis=1)

    wc = jnp.concatenate([W_cat, W_sent], axis=1)
    bc = jnp.concatenate([b_cat, b_sent], axis=0).reshape(1, 16)
    out = _mlp(asp72, opn72, W1, b1.reshape(1, D), wc, bc)

    pairs = out.reshape(B, NPAD, NPAD, 16)[:, :K + 1, :K + 1, :]
    pairs = pairs.reshape(B, (K + 1) * (K + 1), 16)[:, :-1, :]
    cat_logits = pairs[:, :, :13]
    sent_logits = pairs[:, :, 13:16]
    return asp_topk_scores, opn_topk_scores, cat_logits, sent_logits


# speedup vs baseline: 2.9950x; 1.1529x over previous
import functools

import jax
import jax.numpy as jnp
from jax import lax
from jax.experimental import pallas as pl
from jax.experimental.pallas import tpu as pltpu
from jax.experimental.pallas import tpu_sc as plsc

K = 64
N = 4096
D = 512
B = 8
NPAD = 72
NROW = 2 * B


def _topk_body(s_ref, vals_ref, ids_ref):
    scores = s_ref[...]
    col = lax.broadcasted_iota(jnp.int32, (NROW, N), 1)
    lane_k = lax.broadcasted_iota(jnp.int32, (NROW, K), 1)

    def step(k, carry):
        sc, vals, ids = carry
        m = jnp.max(sc, axis=1, keepdims=True)
        cand = jnp.where(sc >= m, col, jnp.int32(N))
        idx = jnp.min(cand, axis=1, keepdims=True)
        vals = jnp.where(lane_k == k, m, vals)
        ids = jnp.where(lane_k == k, idx, ids)
        sc = jnp.where(col == idx, -jnp.inf, sc)
        return sc, vals, ids

    vals0 = jnp.zeros((NROW, K), jnp.float32)
    ids0 = jnp.zeros((NROW, K), jnp.int32)
    _, vals, ids = lax.fori_loop(0, K, step, (scores, vals0, ids0))
    vals_ref[...] = vals
    row = lax.broadcasted_iota(jnp.int32, (NROW, K), 0)
    ids_ref[...] = ids + (row % B) * N


def _topk(scores_stacked):
    return pl.pallas_call(
        _topk_body,
        out_shape=[
            jax.ShapeDtypeStruct((NROW, K), jnp.float32),
            jax.ShapeDtypeStruct((NROW, K), jnp.int32),
        ],
    )(scores_stacked)


def _sc_gather(table, idx_flat):
    n_idx = NROW * K
    nw = 32
    bpw = n_idx // nw
    mesh = plsc.VectorSubcoreMesh(core_axis_name="c", subcore_axis_name="s")

    @functools.partial(
        pl.kernel,
        mesh=mesh,
        out_type=jax.ShapeDtypeStruct((n_idx, D), jnp.float32),
        scratch_types=[
            pltpu.VMEM((bpw,), jnp.int32),
            pltpu.VMEM((bpw, D), jnp.float32),
            pltpu.SemaphoreType.DMA,
        ],
    )
    def gk(table_hbm, idx_hbm, out_hbm, idx_v, rows_v, sem):
        wid = lax.axis_index("s") * 2 + lax.axis_index("c")
        base = wid * bpw
        pltpu.sync_copy(idx_hbm.at[pl.ds(base, bpw)], idx_v)
        pltpu.async_copy(table_hbm.at[idx_v], rows_v, sem).wait()
        pltpu.sync_copy(rows_v, out_hbm.at[pl.ds(base, bpw)])

    return gk(table, idx_flat)


def _mlp_body(ra_ref, ro_ref, na_ref, no_ref, w1_ref, b1_ref, wc_ref, bc_ref,
              cat_ref, sent_ref):
    ra = ra_ref[0]
    ro = ro_ref[0]
    w1a = w1_ref[:D, :]
    w1b = w1_ref[D:, :]
    pa = jnp.dot(ra, w1a, preferred_element_type=jnp.float32)
    pan = jnp.dot(na_ref[0], w1a, preferred_element_type=jnp.float32)
    o65 = jnp.concatenate([ro, no_ref[0]], axis=0)
    po = jnp.dot(o65, w1b, preferred_element_type=jnp.float32)
    po = po + b1_ref[...]
    wc = wc_ref[...]
    bc = bc_ref[...]
    for i in range(K):
        h = jax.nn.relu(pa[i:i + 1, :] + po)
        out = jnp.dot(h, wc, preferred_element_type=jnp.float32) + bc
        cat_ref[0, i * 65:(i + 1) * 65, :] = out[:, :13]
        sent_ref[0, i * 65:(i + 1) * 65, :] = out[:, 13:16]
    h = jax.nn.relu(pan + po[:K, :])
    out = jnp.dot(h, wc, preferred_element_type=jnp.float32) + bc
    cat_ref[0, K * 65:K * 65 + K, :] = out[:, :13]
    sent_ref[0, K * 65:K * 65 + K, :] = out[:, 13:16]


def _mlp(rows16, null_asp, null_opn, w1, b1r, wc, bcr):
    npairs = (K + 1) * (K + 1) - 1
    return pl.pallas_call(
        _mlp_body,
        grid=(B,),
        in_specs=[
            pl.BlockSpec((1, K, D), lambda b: (b, 0, 0)),
            pl.BlockSpec((1, K, D), lambda b: (b + B, 0, 0)),
            pl.BlockSpec((1, 1, D), lambda b: (b, 0, 0)),
            pl.BlockSpec((1, 1, D), lambda b: (b, 0, 0)),
            pl.BlockSpec((2 * D, D), lambda b: (0, 0)),
            pl.BlockSpec((1, D), lambda b: (0, 0)),
            pl.BlockSpec((D, 16), lambda b: (0, 0)),
            pl.BlockSpec((1, 16), lambda b: (0, 0)),
        ],
        out_specs=[
            pl.BlockSpec((1, npairs, 13), lambda b: (b, 0, 0)),
            pl.BlockSpec((1, npairs, 3), lambda b: (b, 0, 0)),
        ],
        out_shape=[
            jax.ShapeDtypeStruct((B, npairs, 13), jnp.float32),
            jax.ShapeDtypeStruct((B, npairs, 3), jnp.float32),
        ],
    )(rows16, rows16, null_asp.reshape(B, 1, D), null_opn.reshape(B, 1, D),
      w1, b1r, wc, bcr)


def kernel(asp_scores, opn_scores, span_reprs, null_asp_repr, null_opn_repr,
           W1, b1, W_cat, b_cat, W_sent, b_sent):
    scores = jnp.concatenate([asp_scores, opn_scores], axis=0)
    vals, ids = _topk(scores)
    asp_topk_scores = vals[:B]
    opn_topk_scores = vals[B:]
    table = span_reprs.reshape(B * N, D)
    rows = _sc_gather(table, ids.reshape(-1))
    rows16 = rows.reshape(2 * B, K, D)

    wc = jnp.concatenate([W_cat, W_sent], axis=1)
    bc = jnp.concatenate([b_cat, b_sent], axis=0).reshape(1, 16)
    cat_logits, sent_logits = _mlp(rows16, null_asp_repr, null_opn_repr,
                                   W1, b1.reshape(1, D), wc, bc)
    return asp_topk_scores, opn_topk_scores, cat_logits, sent_logits
